# baseline (speedup 1.0000x reference)
import functools

import jax
import jax.numpy as jnp
from jax import lax
from jax.experimental import pallas as pl
from jax.experimental.pallas import tpu as pltpu
from jax.experimental.pallas import tpu_sc as plsc

L = 16
PCTS = (0.25, 0.5, 0.75)


def _tc_quantiles(x, n):
    col = lax.broadcasted_iota(jnp.int32, x.shape, 1)
    cdf = x
    sh = 1
    while sh < n:
        cdf = cdf + jnp.where(col >= sh, jnp.roll(cdf, sh, 1), 0.0)
        sh *= 2
    cdf0 = cdf[:, :1]
    qs = []
    for p in PCTS:
        lt = cdf < p
        idx = jnp.sum(lt.astype(jnp.float32), axis=1, keepdims=True)
        yb = jnp.min(jnp.where(lt, jnp.float32(jnp.inf), cdf), axis=1,
                     keepdims=True)
        ya = jnp.max(jnp.where(lt, cdf, 0.0), axis=1, keepdims=True)
        notf = idx >= n
        idx = jnp.where(notf, 0.0, idx)
        yb = jnp.where(notf, cdf0, yb)
        ya = jnp.where(notf, 0.0, ya)
        qs.append(idx + 1.0 + (p - yb) / (yb - ya))
    return qs


def _deep_tc_body(e_ref, t_ref, o_ref):
    n = e_ref.shape[1]
    qe = _tc_quantiles(e_ref[...], n)
    qt = _tc_quantiles(t_ref[...], n)
    s = jnp.zeros_like(qe[0])
    for a, b in zip(qe, qt):
        s = s + jnp.abs(a - b)
    o_ref[...] = jnp.broadcast_to(s, o_ref.shape)


def _deep_tc_loss(p_estimate, p_target):
    B, N = p_estimate.shape
    R = 256
    out = pl.pallas_call(
        _deep_tc_body,
        grid=(B // R,),
        in_specs=[pl.BlockSpec((R, N), lambda i: (i, 0)),
                  pl.BlockSpec((R, N), lambda i: (i, 0))],
        out_specs=pl.BlockSpec((R, 128), lambda i: (i, 0)),
        out_shape=jax.ShapeDtypeStruct((B, 128), jnp.float32),
    )(p_estimate, p_target)
    return jnp.sum(out[:, 0]) / jnp.float32(B * len(PCTS))


def _scan_fast(bufs, cbufs, g):
    lane = lax.iota(jnp.int32, L)
    lane16 = lane * L
    zi = jnp.zeros((L,), jnp.int32)

    def col_hot(cdf_ref, c, x, j):
        csum, c0, c1, c2 = c
        new = csum + x
        cdf_ref[j, :] = new
        c0 = c0 + (new < PCTS[0]).astype(jnp.int32)
        c1 = c1 + (new < PCTS[1]).astype(jnp.int32)
        c2 = c2 + (new < PCTS[2]).astype(jnp.int32)
        return (new, c0, c1, c2)

    def col_mid(cdf_ref, c, x, j):
        csum, c0, c1, c2 = c
        new = csum + x
        plsc.store_scatter(cdf_ref, [jnp.full((L,), j, jnp.int32), lane], new)
        c0 = c0 + (new < PCTS[0]).astype(jnp.int32)
        c1 = c1 + (new < PCTS[1]).astype(jnp.int32)
        c2 = c2 + (new < PCTS[2]).astype(jnp.int32)
        return (new, c0, c1, c2)

    c = ((jnp.zeros((L,), jnp.float32), zi, zi, zi),
         (jnp.zeros((L,), jnp.float32), zi, zi, zi))
    for j in range(8):
        flat = g * (L * L) + j + lane16
        r, cl = lax.shift_right_logical(flat, 7), lax.bitwise_and(flat, 127)
        xe = plsc.load_gather(bufs[0], [r, cl])
        xt = plsc.load_gather(bufs[1], [r, cl])
        c = (col_hot(cbufs[0], c[0], xe, j), col_hot(cbufs[1], c[1], xt, j))

    def mid(c):
        def body(j, c):
            flat = g * (L * L) + j + lane16
            r = lax.shift_right_logical(flat, 7)
            cl = lax.bitwise_and(flat, 127)
            xe = plsc.load_gather(bufs[0], [r, cl])
            xt = plsc.load_gather(bufs[1], [r, cl])
            return (col_mid(cbufs[0], c[0], xe, j),
                    col_mid(cbufs[1], c[1], xt, j))
        return lax.fori_loop(8, L, body, c)

    done = jnp.logical_and(jnp.all(c[0][0] >= PCTS[2]),
                           jnp.all(c[1][0] >= PCTS[2]))
    c = lax.cond(done, lambda c: c, mid, c)

    out = []
    for (_, c0, c1, c2), cbuf in zip(c, cbufs):
        bad = c2 >= L
        qs = []
        for p, cnt in ((PCTS[0], c0), (PCTS[1], c1), (PCTS[2], c2)):
            cc = jnp.minimum(cnt, L - 1)
            yb = plsc.load_gather(cbuf, [cc, lane])
            ya_prev = plsc.load_gather(cbuf, [jnp.maximum(cc - 1, 0), lane])
            ya = jnp.where(cnt == 0, jnp.zeros((L,), jnp.float32), ya_prev)
            qs.append(cnt.astype(jnp.float32) + 1.0 + (p - yb) / (yb - ya))
        out.append((qs, bad))
    return out


def _make_fast_call(B, N):
    rw = B // 32
    ng = rw // L
    srows = rw * L // 128
    mesh = plsc.VectorSubcoreMesh(core_axis_name="c", subcore_axis_name="s")

    @functools.partial(
        pl.kernel,
        mesh=mesh,
        out_type=jax.ShapeDtypeStruct((2, 32, L), jnp.float32),
        scratch_types=[
            pltpu.VMEM((srows, 128), jnp.float32),
            pltpu.VMEM((srows, 128), jnp.float32),
            pltpu.VMEM((L, L), jnp.float32),
            pltpu.VMEM((L, L), jnp.float32),
            pltpu.VMEM((L,), jnp.float32),
            pltpu.VMEM((L,), jnp.float32),
            pltpu.SemaphoreType.DMA,
        ],
        compiler_params=pltpu.CompilerParams(needs_layout_passes=False),
    )
    def k(e16, t16, out, ebuf, tbuf, ecdf, tcdf, accv, badv, sem):
        wid = lax.axis_index("c") * 16 + lax.axis_index("s")
        cp1 = pltpu.async_copy(e16.at[pl.ds(wid * srows, srows), :], ebuf,
                               sem)
        cp2 = pltpu.async_copy(t16.at[pl.ds(wid * srows, srows), :], tbuf,
                               sem)
        cp1.wait()
        cp2.wait()

        def group(g, carry):
            acc, badf = carry
            (qe, bade), (qt, badt) = _scan_fast((ebuf, tbuf), (ecdf, tcdf), g)
            badrow = jnp.logical_or(badt, bade)
            s = jnp.zeros((L,), jnp.float32)
            for qti, qei in zip(qt, qe):
                s = s + jnp.abs(qti - qei)
            zf = jnp.zeros((L,), jnp.float32)
            acc = acc + jnp.where(badrow, zf, s)
            badf = badf + jnp.where(badrow, jnp.ones((L,), jnp.float32), zf)
            return acc, badf

        acc, badf = lax.fori_loop(
            0, ng, group,
            (jnp.zeros((L,), jnp.float32), jnp.zeros((L,), jnp.float32)))
        accv[...] = acc
        pltpu.sync_copy(accv, out.at[0, wid])
        badv[...] = badf
        pltpu.sync_copy(badv, out.at[1, wid])

    return k


@jax.jit
def kernel(p_estimate, p_target):
    B, N = p_estimate.shape
    denom = jnp.float32(B * len(PCTS))
    e16 = p_estimate[:, :L].reshape(B * L // 128, 128)
    t16 = p_target[:, :L].reshape(B * L // 128, 128)

    part = _make_fast_call(B, N)(e16, t16)
    nbad = jnp.sum(part[1])

    def deep(_):
        return _deep_tc_loss(p_estimate, p_target)

    return lax.cond(nbad > 0, deep,
                    lambda _: jnp.sum(part[0]) / denom, None)


# speedup vs baseline: 1.0008x; 1.0008x over previous
import functools

import jax
import jax.numpy as jnp
from jax import lax
from jax.experimental import pallas as pl
from jax.experimental.pallas import tpu as pltpu
from jax.experimental.pallas import tpu_sc as plsc

L = 16
PCTS = (0.25, 0.5, 0.75)


def _tc_quantiles(x, n):
    col = lax.broadcasted_iota(jnp.int32, x.shape, 1)
    cdf = x
    sh = 1
    while sh < n:
        cdf = cdf + jnp.where(col >= sh, jnp.roll(cdf, sh, 1), 0.0)
        sh *= 2
    cdf0 = cdf[:, :1]
    qs = []
    for p in PCTS:
        lt = cdf < p
        idx = jnp.sum(lt.astype(jnp.float32), axis=1, keepdims=True)
        yb = jnp.min(jnp.where(lt, jnp.float32(jnp.inf), cdf), axis=1,
                     keepdims=True)
        ya = jnp.max(jnp.where(lt, cdf, 0.0), axis=1, keepdims=True)
        notf = idx >= n
        idx = jnp.where(notf, 0.0, idx)
        yb = jnp.where(notf, cdf0, yb)
        ya = jnp.where(notf, 0.0, ya)
        qs.append(idx + 1.0 + (p - yb) / (yb - ya))
    return qs


def _deep_tc_body(e_ref, t_ref, o_ref):
    n = e_ref.shape[1]
    qe = _tc_quantiles(e_ref[...], n)
    qt = _tc_quantiles(t_ref[...], n)
    s = jnp.zeros_like(qe[0])
    for a, b in zip(qe, qt):
        s = s + jnp.abs(a - b)
    o_ref[...] = jnp.broadcast_to(s, o_ref.shape)


def _deep_tc_loss(p_estimate, p_target):
    B, N = p_estimate.shape
    R = 256
    out = pl.pallas_call(
        _deep_tc_body,
        grid=(B // R,),
        in_specs=[pl.BlockSpec((R, N), lambda i: (i, 0)),
                  pl.BlockSpec((R, N), lambda i: (i, 0))],
        out_specs=pl.BlockSpec((R, 128), lambda i: (i, 0)),
        out_shape=jax.ShapeDtypeStruct((B, 128), jnp.float32),
    )(p_estimate, p_target)
    return jnp.sum(out[:, 0]) / jnp.float32(B * len(PCTS))


def _scan_fast(bufs, cbufs, g):
    lane = lax.iota(jnp.int32, L)
    lane16 = lane * L
    zi = jnp.zeros((L,), jnp.int32)

    def col_hot(cdf_ref, c, x, j):
        csum, c0, c1, c2 = c
        new = csum + x
        cdf_ref[j, :] = new
        c0 = c0 + (new < PCTS[0]).astype(jnp.int32)
        c1 = c1 + (new < PCTS[1]).astype(jnp.int32)
        c2 = c2 + (new < PCTS[2]).astype(jnp.int32)
        return (new, c0, c1, c2)

    def col_mid(cdf_ref, c, x, j):
        csum, c0, c1, c2 = c
        new = csum + x
        plsc.store_scatter(cdf_ref, [jnp.full((L,), j, jnp.int32), lane], new)
        c0 = c0 + (new < PCTS[0]).astype(jnp.int32)
        c1 = c1 + (new < PCTS[1]).astype(jnp.int32)
        c2 = c2 + (new < PCTS[2]).astype(jnp.int32)
        return (new, c0, c1, c2)

    c = ((jnp.zeros((L,), jnp.float32), zi, zi, zi),
         (jnp.zeros((L,), jnp.float32), zi, zi, zi))
    for j in range(8):
        flat = g * (L * L) + j + lane16
        r, cl = lax.shift_right_logical(flat, 7), lax.bitwise_and(flat, 127)
        xe = plsc.load_gather(bufs[0], [r, cl])
        xt = plsc.load_gather(bufs[1], [r, cl])
        c = (col_hot(cbufs[0], c[0], xe, j), col_hot(cbufs[1], c[1], xt, j))

    def mid(c):
        def body(j, c):
            flat = g * (L * L) + j + lane16
            r = lax.shift_right_logical(flat, 7)
            cl = lax.bitwise_and(flat, 127)
            xe = plsc.load_gather(bufs[0], [r, cl])
            xt = plsc.load_gather(bufs[1], [r, cl])
            return (col_mid(cbufs[0], c[0], xe, j),
                    col_mid(cbufs[1], c[1], xt, j))
        return lax.fori_loop(8, L, body, c)

    done = jnp.logical_and(jnp.all(c[0][0] >= PCTS[2]),
                           jnp.all(c[1][0] >= PCTS[2]))
    c = lax.cond(done, lambda c: c, mid, c)

    out = []
    for (_, c0, c1, c2), cbuf in zip(c, cbufs):
        bad = c2 >= L
        qs = []
        for p, cnt in ((PCTS[0], c0), (PCTS[1], c1), (PCTS[2], c2)):
            cc = jnp.minimum(cnt, L - 1)
            yb = plsc.load_gather(cbuf, [cc, lane])
            ya_prev = plsc.load_gather(cbuf, [jnp.maximum(cc - 1, 0), lane])
            ya = jnp.where(cnt == 0, jnp.zeros((L,), jnp.float32), ya_prev)
            qs.append(cnt.astype(jnp.float32) + 1.0 + (p - yb) / (yb - ya))
        out.append((qs, bad))
    return out


def _make_fast_call(B, N):
    rw = B // 32
    ng = rw // L
    srows = rw * L // 128
    mesh = plsc.VectorSubcoreMesh(core_axis_name="c", subcore_axis_name="s")

    @functools.partial(
        pl.kernel,
        mesh=mesh,
        out_type=(jax.ShapeDtypeStruct((32, L), jnp.float32),
                  jax.ShapeDtypeStruct((32, L), jnp.float32)),
        scratch_types=[
            pltpu.VMEM((srows, 128), jnp.float32),
            pltpu.VMEM((srows, 128), jnp.float32),
            pltpu.VMEM((L, L), jnp.float32),
            pltpu.VMEM((L, L), jnp.float32),
            pltpu.VMEM((L,), jnp.float32),
            pltpu.VMEM((L,), jnp.float32),
            pltpu.SemaphoreType.DMA,
        ],
        compiler_params=pltpu.CompilerParams(needs_layout_passes=False),
    )
    def k(e16, t16, outa, outb, ebuf, tbuf, ecdf, tcdf, accv, badv, sem):
        wid = lax.axis_index("c") * 16 + lax.axis_index("s")
        cp1 = pltpu.async_copy(e16.at[pl.ds(wid * srows, srows), :], ebuf,
                               sem)
        cp2 = pltpu.async_copy(t16.at[pl.ds(wid * srows, srows), :], tbuf,
                               sem)
        cp1.wait()
        cp2.wait()

        def group(g, carry):
            acc, badf = carry
            (qe, bade), (qt, badt) = _scan_fast((ebuf, tbuf), (ecdf, tcdf), g)
            badrow = jnp.logical_or(badt, bade)
            s = jnp.zeros((L,), jnp.float32)
            for qti, qei in zip(qt, qe):
                s = s + jnp.abs(qti - qei)
            zf = jnp.zeros((L,), jnp.float32)
            acc = acc + jnp.where(badrow, zf, s)
            badf = badf + jnp.where(badrow, jnp.ones((L,), jnp.float32), zf)
            return acc, badf

        acc, badf = lax.fori_loop(
            0, ng, group,
            (jnp.zeros((L,), jnp.float32), jnp.zeros((L,), jnp.float32)))
        accv[...] = acc
        pltpu.sync_copy(accv, outa.at[wid])
        badv[...] = badf
        pltpu.sync_copy(badv, outb.at[wid])

    return k


@jax.jit
def kernel(p_estimate, p_target):
    B, N = p_estimate.shape
    denom = jnp.float32(B * len(PCTS))
    e16 = p_estimate[:, :L].reshape(B * L // 128, 128)
    t16 = p_target[:, :L].reshape(B * L // 128, 128)

    psum, pbad = _make_fast_call(B, N)(e16, t16)
    nbad = jnp.sum(pbad)

    def deep(_):
        return _deep_tc_loss(p_estimate, p_target)

    return lax.cond(nbad > 0, deep,
                    lambda _: jnp.sum(psum) / denom, None)
